# baseline (speedup 1.0000x reference)
import jax
import jax.numpy as jnp
from jax.experimental import pallas as pl
from jax.experimental.pallas import tpu as pltpu

N_ROWS = 16384
D = 1024
BLK = 1024
NBLK = N_ROWS // BLK
NBUF = 4


def _fused_body(a_hbm, w_ref, o_ref, bufs, scores_v, sems):
    for i in range(NBUF):
        pltpu.make_async_copy(
            a_hbm.at[pl.ds(i * BLK, BLK)], bufs.at[i], sems.at[i]
        ).start()
    w = w_ref[...]
    for i in range(NBLK):
        b = i % NBUF
        pltpu.make_async_copy(
            a_hbm.at[pl.ds(i * BLK, BLK)], bufs.at[b], sems.at[b]
        ).wait()
        scores_v[i, :] = jnp.sum(bufs[b] * w, axis=1)
        nxt = i + NBUF
        if nxt < NBLK:
            pltpu.make_async_copy(
                a_hbm.at[pl.ds(nxt * BLK, BLK)], bufs.at[b], sems.at[b]
            ).start()
    sc = scores_v[...]
    m = jnp.max(sc)
    e = jnp.exp(sc - m)
    o_ref[...] = e * (1.0 / jnp.sum(e))


def kernel(feature_vector, W):
    probs = pl.pallas_call(
        _fused_body,
        in_specs=[
            pl.BlockSpec(memory_space=pl.ANY),
            pl.BlockSpec((1, D), lambda: (0, 0)),
        ],
        out_specs=pl.BlockSpec((NBLK, BLK), lambda: (0, 0)),
        out_shape=jax.ShapeDtypeStruct((NBLK, BLK), jnp.float32),
        scratch_shapes=[
            pltpu.VMEM((NBUF, BLK, D), jnp.float32),
            pltpu.VMEM((NBLK, BLK), jnp.float32),
            pltpu.SemaphoreType.DMA((NBUF,)),
        ],
    )(feature_vector, W.reshape(1, D))
    return probs.reshape(1, N_ROWS, 1)


# speedup vs baseline: 1.1096x; 1.1096x over previous
import jax
import jax.numpy as jnp
from jax.experimental import pallas as pl
from jax.experimental.pallas import tpu as pltpu

N_ROWS = 16384
D = 1024
SUB = 8
BLK = SUB * 128
NBLK = N_ROWS // BLK
NBUF = 4


def _fused_body(a_hbm, w_ref, o_ref, bufs, scores_v, sems):
    for i in range(NBUF):
        pltpu.make_async_copy(
            a_hbm.at[pl.ds(i * SUB, SUB)], bufs.at[i], sems.at[i]
        ).start()
    w = w_ref[...]
    for i in range(NBLK):
        b = i % NBUF
        pltpu.make_async_copy(
            a_hbm.at[pl.ds(i * SUB, SUB)], bufs.at[b], sems.at[b]
        ).wait()
        scores_v[pl.ds(i * SUB, SUB), :] = jnp.sum(bufs[b] * w, axis=2)
        nxt = i + NBUF
        if nxt < NBLK:
            pltpu.make_async_copy(
                a_hbm.at[pl.ds(nxt * SUB, SUB)], bufs.at[b], sems.at[b]
            ).start()
    sc = scores_v[...]
    m = jnp.max(sc)
    e = jnp.exp(sc - m)
    o_ref[...] = e * (1.0 / jnp.sum(e))


def kernel(feature_vector, W):
    a3 = feature_vector.reshape(128, 128, D)
    probs = pl.pallas_call(
        _fused_body,
        in_specs=[
            pl.BlockSpec(memory_space=pl.ANY),
            pl.BlockSpec((1, 1, D), lambda: (0, 0, 0)),
        ],
        out_specs=pl.BlockSpec((128, 128), lambda: (0, 0)),
        out_shape=jax.ShapeDtypeStruct((128, 128), jnp.float32),
        scratch_shapes=[
            pltpu.VMEM((NBUF, SUB, 128, D), jnp.float32),
            pltpu.VMEM((128, 128), jnp.float32),
            pltpu.SemaphoreType.DMA((NBUF,)),
        ],
    )(a3, W.reshape(1, 1, D))
    return probs.reshape(1, N_ROWS, 1)
